# baseline (speedup 1.0000x reference)
import functools

import jax
import jax.numpy as jnp
from jax import lax
from jax.experimental import pallas as pl
from jax.experimental.pallas import tpu as pltpu
from jax.experimental.pallas import tpu_sc as plsc

VOCAB = 100000
D_MODEL = 768
BATCH = 4
SEQ = 8192


def _make_embed(vocab, d, batch, seq, rows_per_chunk, num_cores=2,
                num_subcores=16, interpret=False):
  n_rows = batch * seq
  nw = num_cores * num_subcores
  assert n_rows % nw == 0
  rows_per_w = n_rows // nw
  assert rows_per_w % rows_per_chunk == 0
  n_chunks = rows_per_w // rows_per_chunk
  lanes_per_row = d // 16
  assert d % 16 == 0

  mesh = plsc.VectorSubcoreMesh(core_axis_name="c", subcore_axis_name="s",
                                num_cores=num_cores,
                                num_subcores=num_subcores)

  @functools.partial(
      pl.kernel,
      out_type=jax.ShapeDtypeStruct((n_rows, d), jnp.float32),
      mesh=mesh,
      scratch_types=[
          pltpu.VMEM((rows_per_w,), jnp.int32),
          pltpu.VMEM((rows_per_chunk, d), jnp.float32),
          pltpu.VMEM((rows_per_chunk, d), jnp.float32),
          pltpu.SemaphoreType.DMA,
          pltpu.SemaphoreType.DMA,
      ],
      interpret=interpret,
  )
  def embed(x_hbm, tok_hbm, pos_hbm, out_hbm, idx_v, tok_v, pos_v, sem, psem):
    wid = lax.axis_index("s") * num_cores + lax.axis_index("c")
    base = wid * rows_per_w
    pos_base = base % seq
    pltpu.sync_copy(x_hbm.at[pl.ds(base, rows_per_w)], idx_v)

    def chunk(g, carry):
      off = g * rows_per_chunk
      cp_tok = pltpu.async_copy(
          tok_hbm.at[idx_v.at[pl.ds(off, rows_per_chunk)]], tok_v, sem)
      cp_pos = pltpu.async_copy(
          pos_hbm.at[pl.ds(pos_base + off, rows_per_chunk)], pos_v, psem)
      cp_tok.wait()
      cp_pos.wait()

      def add(k, c2):
        i = k // lanes_per_row
        j = (k % lanes_per_row) * 16
        tok_v[i, pl.ds(j, 16)] = tok_v[i, pl.ds(j, 16)] + pos_v[i, pl.ds(j, 16)]
        return c2

      lax.fori_loop(0, rows_per_chunk * lanes_per_row, add, 0, unroll=4)
      pltpu.sync_copy(tok_v, out_hbm.at[pl.ds(base + off, rows_per_chunk)])
      return carry

    lax.fori_loop(0, n_chunks, chunk, 0)

  return embed


_embed_full = _make_embed(VOCAB, D_MODEL, BATCH, SEQ, rows_per_chunk=64)


@jax.jit
def kernel(x, token_table, pos_table):
  xf = x.reshape(-1).astype(jnp.int32)
  out = _embed_full(xf, token_table, pos_table)
  return out.reshape(BATCH, SEQ, D_MODEL)


# speedup vs baseline: 2.9206x; 2.9206x over previous
import functools

import jax
import jax.numpy as jnp
from jax import lax
from jax.experimental import pallas as pl
from jax.experimental.pallas import tpu as pltpu
from jax.experimental.pallas import tpu_sc as plsc

VOCAB = 100000
D_MODEL = 768
BATCH = 4
SEQ = 8192


def _make_embed(vocab, d, batch, seq, rows_per_chunk, num_cores=2,
                num_subcores=16):
  n_rows = batch * seq
  nw = num_cores * num_subcores
  assert n_rows % nw == 0
  rows_per_w = n_rows // nw
  assert rows_per_w % (2 * rows_per_chunk) == 0
  n_chunks = rows_per_w // rows_per_chunk
  n_pairs = n_chunks // 2
  lanes_per_row = d // 16
  assert d % 16 == 0

  mesh = plsc.VectorSubcoreMesh(core_axis_name="c", subcore_axis_name="s",
                                num_cores=num_cores,
                                num_subcores=num_subcores)

  @functools.partial(
      pl.kernel,
      out_type=jax.ShapeDtypeStruct((n_rows, d), jnp.float32),
      mesh=mesh,
      scratch_types=[
          pltpu.VMEM((rows_per_w,), jnp.int32),
          pltpu.VMEM((rows_per_chunk, d), jnp.float32),
          pltpu.VMEM((rows_per_chunk, d), jnp.float32),
          pltpu.VMEM((rows_per_chunk, d), jnp.float32),
          pltpu.VMEM((rows_per_chunk, d), jnp.float32),
          pltpu.SemaphoreType.DMA,
          pltpu.SemaphoreType.DMA,
          pltpu.SemaphoreType.DMA,
          pltpu.SemaphoreType.DMA,
      ],
  )
  def embed(x_hbm, tok_hbm, pos_hbm, out_hbm, idx_v,
            tok0, tok1, pos0, pos1, ts0, ts1, ps0, ps1):
    wid = lax.axis_index("s") * num_cores + lax.axis_index("c")
    base = wid * rows_per_w
    pos_base = base % seq
    pltpu.sync_copy(x_hbm.at[pl.ds(base, rows_per_w)], idx_v)

    toks = (tok0, tok1)
    poss = (pos0, pos1)
    tsems = (ts0, ts1)
    psems = (ps0, ps1)

    def start_in(g, slot):
      off = g * rows_per_chunk
      pltpu.async_copy(
          tok_hbm.at[idx_v.at[pl.ds(off, rows_per_chunk)]], toks[slot],
          tsems[slot])
      pltpu.async_copy(
          pos_hbm.at[pl.ds(pos_base + off, rows_per_chunk)], poss[slot],
          psems[slot])

    def wait_in(g, slot):
      off = g * rows_per_chunk
      pltpu.make_async_copy(
          tok_hbm.at[idx_v.at[pl.ds(off, rows_per_chunk)]], toks[slot],
          tsems[slot]).wait()
      pltpu.make_async_copy(
          pos_hbm.at[pl.ds(pos_base + off, rows_per_chunk)], poss[slot],
          psems[slot]).wait()

    def compute(slot):
      tok_v = toks[slot]
      pos_v = poss[slot]

      def add_row(r, _):
        for j in range(lanes_per_row):
          o = j * 16
          tok_v[r, pl.ds(o, 16)] = tok_v[r, pl.ds(o, 16)] + pos_v[r, pl.ds(o, 16)]
        return 0

      lax.fori_loop(0, rows_per_chunk, add_row, 0)

    def write_out(g, slot):
      off = g * rows_per_chunk
      pltpu.sync_copy(toks[slot], out_hbm.at[pl.ds(base + off, rows_per_chunk)])

    start_in(0, 0)

    def pair(p, carry):
      g0 = 2 * p
      g1 = g0 + 1
      start_in(g1, 1)
      wait_in(g0, 0)
      compute(0)
      write_out(g0, 0)

      @pl.when(g1 + 1 < n_chunks)
      def _():
        start_in(g1 + 1, 0)

      wait_in(g1, 1)
      compute(1)
      write_out(g1, 1)
      return carry

    lax.fori_loop(0, n_pairs, pair, 0)

  return embed


_embed_full = _make_embed(VOCAB, D_MODEL, BATCH, SEQ, rows_per_chunk=32)


@jax.jit
def kernel(x, token_table, pos_table):
  xf = x.reshape(-1).astype(jnp.int32)
  out = _embed_full(xf, token_table, pos_table)
  return out.reshape(BATCH, SEQ, D_MODEL)


# speedup vs baseline: 2.9568x; 1.0124x over previous
import functools

import jax
import jax.numpy as jnp
from jax import lax
from jax.experimental import pallas as pl
from jax.experimental.pallas import tpu as pltpu
from jax.experimental.pallas import tpu_sc as plsc

VOCAB = 100000
D_MODEL = 768
BATCH = 4
SEQ = 8192

_NSLOTS = 4


def _make_embed(vocab, d, batch, seq, rows_per_chunk, num_cores=2,
                num_subcores=16):
  n_rows = batch * seq
  nw = num_cores * num_subcores
  assert n_rows % nw == 0
  rows_per_w = n_rows // nw
  assert rows_per_w % (_NSLOTS * rows_per_chunk) == 0
  n_chunks = rows_per_w // rows_per_chunk
  n_rounds = n_chunks // _NSLOTS
  lanes_per_row = d // 16
  assert d % 16 == 0

  mesh = plsc.VectorSubcoreMesh(core_axis_name="c", subcore_axis_name="s",
                                num_cores=num_cores,
                                num_subcores=num_subcores)

  @functools.partial(
      pl.kernel,
      out_type=jax.ShapeDtypeStruct((n_rows, d), jnp.float32),
      mesh=mesh,
      scratch_types=[
          pltpu.VMEM((rows_per_w,), jnp.int32),
          [pltpu.VMEM((rows_per_chunk, d), jnp.float32)] * _NSLOTS,
          [pltpu.VMEM((rows_per_chunk, d), jnp.float32)] * _NSLOTS,
          [pltpu.SemaphoreType.DMA] * _NSLOTS,
          [pltpu.SemaphoreType.DMA] * _NSLOTS,
          [pltpu.SemaphoreType.DMA] * _NSLOTS,
      ],
  )
  def embed(x_hbm, tok_hbm, pos_hbm, out_hbm, idx_v,
            toks, poss, tsems, psems, osems):
    wid = lax.axis_index("s") * num_cores + lax.axis_index("c")
    base = wid * rows_per_w
    pos_base = base % seq
    pltpu.sync_copy(x_hbm.at[pl.ds(base, rows_per_w)], idx_v)

    def start_in(g, slot):
      off = g * rows_per_chunk
      pltpu.async_copy(
          tok_hbm.at[idx_v.at[pl.ds(off, rows_per_chunk)]], toks[slot],
          tsems[slot])
      pltpu.async_copy(
          pos_hbm.at[pl.ds(pos_base + off, rows_per_chunk)], poss[slot],
          psems[slot])

    def wait_in(g, slot):
      off = g * rows_per_chunk
      pltpu.make_async_copy(
          tok_hbm.at[idx_v.at[pl.ds(off, rows_per_chunk)]], toks[slot],
          tsems[slot]).wait()
      pltpu.make_async_copy(
          pos_hbm.at[pl.ds(pos_base + off, rows_per_chunk)], poss[slot],
          psems[slot]).wait()

    def compute(slot):
      tok_v = toks[slot]
      pos_v = poss[slot]

      def add_row(r, carry):
        for j in range(lanes_per_row):
          o = j * 16
          tok_v[r, pl.ds(o, 16)] = tok_v[r, pl.ds(o, 16)] + pos_v[r, pl.ds(o, 16)]
        return carry

      lax.fori_loop(0, rows_per_chunk, add_row, 0)

    def start_out(g, slot):
      off = g * rows_per_chunk
      pltpu.async_copy(
          toks[slot], out_hbm.at[pl.ds(base + off, rows_per_chunk)],
          osems[slot])

    def wait_out(g, slot):
      off = g * rows_per_chunk
      pltpu.make_async_copy(
          toks[slot], out_hbm.at[pl.ds(base + off, rows_per_chunk)],
          osems[slot]).wait()

    for b in range(_NSLOTS - 1):
      start_in(b, b)

    def round_(q, carry):
      for b in range(_NSLOTS):
        g = q * _NSLOTS + b
        wait_in(g, b)
        compute(b)
        start_out(g, b)
        s = (b + _NSLOTS - 1) % _NSLOTS
        gc = g + _NSLOTS - 1

        @pl.when(gc < n_chunks)
        def _():
          @pl.when(gc >= _NSLOTS)
          def _():
            wait_out(gc - _NSLOTS, s)

          start_in(gc, s)

      return carry

    lax.fori_loop(0, n_rounds, round_, 0)

    for b in range(_NSLOTS):
      wait_out(n_chunks - _NSLOTS + b, b)

  return embed


_embed_full = _make_embed(VOCAB, D_MODEL, BATCH, SEQ, rows_per_chunk=16)


@jax.jit
def kernel(x, token_table, pos_table):
  xf = x.reshape(-1).astype(jnp.int32)
  out = _embed_full(xf, token_table, pos_table)
  return out.reshape(BATCH, SEQ, D_MODEL)


# speedup vs baseline: 2.9585x; 1.0006x over previous
import functools

import jax
import jax.numpy as jnp
from jax import lax
from jax.experimental import pallas as pl
from jax.experimental.pallas import tpu as pltpu
from jax.experimental.pallas import tpu_sc as plsc

VOCAB = 100000
D_MODEL = 768
BATCH = 4
SEQ = 8192

_NSLOTS = 4


def _make_embed(vocab, d, batch, seq, rows_per_chunk, num_cores=2,
                num_subcores=16):
  n_rows = batch * seq
  nw = num_cores * num_subcores
  assert n_rows % nw == 0
  rows_per_w = n_rows // nw
  assert rows_per_w % (_NSLOTS * rows_per_chunk) == 0
  n_chunks = rows_per_w // rows_per_chunk
  n_rounds = n_chunks // _NSLOTS
  lanes_per_row = d // 16
  assert d % 16 == 0

  mesh = plsc.VectorSubcoreMesh(core_axis_name="c", subcore_axis_name="s",
                                num_cores=num_cores,
                                num_subcores=num_subcores)

  @functools.partial(
      pl.kernel,
      out_type=jax.ShapeDtypeStruct((n_rows, d), jnp.float32),
      mesh=mesh,
      scratch_types=[
          pltpu.VMEM((rows_per_w,), jnp.int32),
          [pltpu.VMEM((rows_per_chunk, d), jnp.float32)] * _NSLOTS,
          [pltpu.VMEM((rows_per_chunk, d), jnp.float32)] * _NSLOTS,
          [pltpu.SemaphoreType.DMA] * _NSLOTS,
          [pltpu.SemaphoreType.DMA] * _NSLOTS,
          [pltpu.SemaphoreType.DMA] * _NSLOTS,
      ],
  )
  def embed(x_hbm, tok_hbm, pos_hbm, out_hbm, idx_v,
            toks, poss, tsems, psems, osems):
    wid = lax.axis_index("s") * num_cores + lax.axis_index("c")
    base = wid * rows_per_w
    pos_base = base % seq
    pltpu.sync_copy(x_hbm.at[pl.ds(base, rows_per_w)], idx_v)

    def start_in(g, slot):
      off = g * rows_per_chunk
      pltpu.async_copy(
          tok_hbm.at[idx_v.at[pl.ds(off, rows_per_chunk)]], toks[slot],
          tsems[slot])
      pltpu.async_copy(
          pos_hbm.at[pl.ds(pos_base + off, rows_per_chunk)], poss[slot],
          psems[slot])

    def wait_in(g, slot):
      off = g * rows_per_chunk
      pltpu.make_async_copy(
          tok_hbm.at[idx_v.at[pl.ds(off, rows_per_chunk)]], toks[slot],
          tsems[slot]).wait()
      pltpu.make_async_copy(
          pos_hbm.at[pl.ds(pos_base + off, rows_per_chunk)], poss[slot],
          psems[slot]).wait()

    def compute(slot):
      tok_v = toks[slot]
      pos_v = poss[slot]

      def add_row(r, carry):
        for j in range(lanes_per_row):
          o = j * 16
          plsc.addupdate(tok_v.at[r, pl.ds(o, 16)], pos_v[r, pl.ds(o, 16)])
        return carry

      lax.fori_loop(0, rows_per_chunk, add_row, 0)

    def start_out(g, slot):
      off = g * rows_per_chunk
      pltpu.async_copy(
          toks[slot], out_hbm.at[pl.ds(base + off, rows_per_chunk)],
          osems[slot])

    def wait_out(g, slot):
      off = g * rows_per_chunk
      pltpu.make_async_copy(
          toks[slot], out_hbm.at[pl.ds(base + off, rows_per_chunk)],
          osems[slot]).wait()

    for b in range(_NSLOTS - 1):
      start_in(b, b)

    def round_(q, carry):
      for b in range(_NSLOTS):
        g = q * _NSLOTS + b
        wait_in(g, b)
        compute(b)
        start_out(g, b)
        s = (b + _NSLOTS - 1) % _NSLOTS
        gc = g + _NSLOTS - 1

        @pl.when(gc < n_chunks)
        def _():
          @pl.when(gc >= _NSLOTS)
          def _():
            wait_out(gc - _NSLOTS, s)

          start_in(gc, s)

      return carry

    lax.fori_loop(0, n_rounds, round_, 0)

    for b in range(_NSLOTS):
      wait_out(n_chunks - _NSLOTS + b, b)

  return embed


_embed_full = _make_embed(VOCAB, D_MODEL, BATCH, SEQ, rows_per_chunk=16)


@jax.jit
def kernel(x, token_table, pos_table):
  xf = x.reshape(-1).astype(jnp.int32)
  out = _embed_full(xf, token_table, pos_table)
  return out.reshape(BATCH, SEQ, D_MODEL)
